# Initial kernel scaffold; baseline (speedup 1.0000x reference)
import jax
import jax.numpy as jnp
from jax.experimental import pallas as pl


def kernel(node_features, edge_features, edge_index, state_attr, shared_node_weights, shared_edge_weights, Wl_e, bl_e, Wg_e, bg_e, Wl_n, bl_n, Wg_n, bg_n):
    raise NotImplementedError("write your pallas kernel here")


# speedup vs baseline: 3.5629x; 3.5629x over previous
import functools

import jax
import jax.numpy as jnp
from jax import lax
from jax.experimental import pallas as pl
from jax.experimental.pallas import tpu as pltpu
from jax.experimental.pallas import tpu_sc as plsc

N_NODES = 10000
N_EDGES = 320000
D = 128

NC = 2
NS = 16
NW = NC * NS
EPW = N_EDGES // NW
CHUNK = 400
SCHUNK = 200
SLAB = 624
TAIL = N_NODES - NS * SLAB

_LOG2 = 0.6931471805599453

def _vector_mesh():
    return plsc.VectorSubcoreMesh(core_axis_name="c", subcore_axis_name="s")


def _gather_body(nodes_hbm, src_hbm, dst_hbm, vi_hbm, vj_hbm, idx_v, rows_v, sem):
    wid = lax.axis_index("s") * NC + lax.axis_index("c")
    base = wid * EPW

    @pl.loop(0, EPW, step=CHUNK)
    def _(off):
        b = base + off
        pltpu.sync_copy(src_hbm.at[pl.ds(b, CHUNK)], idx_v)
        pltpu.async_copy(nodes_hbm.at[idx_v], rows_v, sem).wait()
        pltpu.sync_copy(rows_v, vi_hbm.at[pl.ds(b, CHUNK)])
        pltpu.sync_copy(dst_hbm.at[pl.ds(b, CHUNK)], idx_v)
        pltpu.async_copy(nodes_hbm.at[idx_v], rows_v, sem).wait()
        pltpu.sync_copy(rows_v, vj_hbm.at[pl.ds(b, CHUNK)])


def _sc_gather(node_features, src, dst):
    k = pl.kernel(
        _gather_body,
        out_type=[
            jax.ShapeDtypeStruct((N_EDGES, D), jnp.float32),
            jax.ShapeDtypeStruct((N_EDGES, D), jnp.float32),
        ],
        mesh=_vector_mesh(),
        scratch_types=[
            pltpu.VMEM((CHUNK,), jnp.int32),
            pltpu.VMEM((CHUNK, D), jnp.float32),
            pltpu.SemaphoreType.DMA,
        ],
    )
    return k(node_features, src, dst)


def _scatter_body(msg_hbm, dst_hbm, zeros_hbm, out_hbm, idx_v, rows_v, acc_sh, sem):
    c = lax.axis_index("c")
    s = lax.axis_index("s")
    wid = s * NC + c
    pltpu.sync_copy(zeros_hbm.at[pl.ds(s * SLAB, SLAB)], acc_sh.at[pl.ds(s * SLAB, SLAB)])

    @pl.when(s == NS - 1)
    def _():
        pltpu.sync_copy(zeros_hbm.at[pl.ds(NS * SLAB, TAIL)],
                        acc_sh.at[pl.ds(NS * SLAB, TAIL)])

    plsc.subcore_barrier()
    base = wid * EPW

    @pl.loop(0, EPW, step=SCHUNK)
    def _(off):
        b = base + off
        pltpu.sync_copy(dst_hbm.at[pl.ds(b, SCHUNK)], idx_v)
        pltpu.sync_copy(msg_hbm.at[pl.ds(b, SCHUNK)], rows_v)
        pltpu.sync_copy(rows_v, acc_sh.at[idx_v], add=True)

    plsc.subcore_barrier()
    pltpu.sync_copy(acc_sh.at[pl.ds(s * SLAB, SLAB)],
                    out_hbm.at[c].at[pl.ds(s * SLAB, SLAB)])

    @pl.when(s == NS - 1)
    def _():
        pltpu.sync_copy(acc_sh.at[pl.ds(NS * SLAB, TAIL)],
                        out_hbm.at[c].at[pl.ds(NS * SLAB, TAIL)])


def _sc_scatter(messages, dst, zeros):
    k = pl.kernel(
        _scatter_body,
        out_type=jax.ShapeDtypeStruct((NC, N_NODES, D), jnp.float32),
        mesh=_vector_mesh(),
        scratch_types=[
            pltpu.VMEM((SCHUNK,), jnp.int32),
            pltpu.VMEM((SCHUNK, D), jnp.float32),
            pltpu.VMEM_SHARED((N_NODES, D), jnp.float32),
            pltpu.SemaphoreType.DMA,
        ],
    )
    return k(messages, dst, zeros)


def _mlp_body(vi_ref, vj_ref, e_ref, sew_ref,
              w1e_ref, w2e_ref, w3e_ref, be_ref,
              w1n_ref, w2n_ref, w3n_ref, bn_ref,
              ne_ref, msg_ref):
    vi = vi_ref[...].astype(jnp.bfloat16)
    vj = vj_ref[...].astype(jnp.bfloat16)
    e = e_ref[...]
    e16 = e.astype(jnp.bfloat16)
    pe = (jnp.dot(vi, w1e_ref[...], preferred_element_type=jnp.float32)
          + jnp.dot(e16, w2e_ref[...], preferred_element_type=jnp.float32)
          + jnp.dot(vj, w3e_ref[...], preferred_element_type=jnp.float32)
          + be_ref[...])
    core = jax.nn.softplus(pe[:, :D]) - _LOG2
    gate = jax.nn.sigmoid(pe[:, D:])
    ne = e + core * gate * sew_ref[...]
    ne_ref[...] = ne
    ne16 = ne.astype(jnp.bfloat16)
    pn = (jnp.dot(vi, w1n_ref[...], preferred_element_type=jnp.float32)
          + jnp.dot(vj, w2n_ref[...], preferred_element_type=jnp.float32)
          + jnp.dot(ne16, w3n_ref[...], preferred_element_type=jnp.float32)
          + bn_ref[...])
    msg_ref[...] = (jax.nn.softplus(pn[:, :D]) - _LOG2) * jax.nn.sigmoid(pn[:, D:])


_EB = 1600


def _tc_mlps(vi, vj, e, sew, w1e, w2e, w3e, be, w1n, w2n, w3n, bn):
    grid = (N_EDGES // _EB,)
    row_spec = pl.BlockSpec((_EB, D), lambda i: (i, 0))
    w_spec = pl.BlockSpec((D, 2 * D), lambda i: (0, 0))
    b_spec = pl.BlockSpec((1, 2 * D), lambda i: (0, 0))
    return pl.pallas_call(
        _mlp_body,
        grid=grid,
        in_specs=[row_spec, row_spec, row_spec, row_spec,
                  w_spec, w_spec, w_spec, b_spec,
                  w_spec, w_spec, w_spec, b_spec],
        out_specs=[row_spec, row_spec],
        out_shape=[
            jax.ShapeDtypeStruct((N_EDGES, D), jnp.float32),
            jax.ShapeDtypeStruct((N_EDGES, D), jnp.float32),
        ],
    )(vi, vj, e, sew, w1e, w2e, w3e, be, w1n, w2n, w3n, bn)


def _axpy_body(n_ref, snw_ref, a0_ref, a1_ref, o_ref):
    o_ref[...] = n_ref[...] + (a0_ref[...] + a1_ref[...]) * snw_ref[...]


def _tc_axpy(nodes, snw, acc0, acc1):
    blk = 1000
    spec = pl.BlockSpec((blk, D), lambda i: (i, 0))
    return pl.pallas_call(
        _axpy_body,
        grid=(N_NODES // blk,),
        in_specs=[spec, spec, spec, spec],
        out_specs=spec,
        out_shape=jax.ShapeDtypeStruct((N_NODES, D), jnp.float32),
    )(nodes, snw, acc0, acc1)


def kernel(node_features, edge_features, edge_index, state_attr,
           shared_node_weights, shared_edge_weights,
           Wl_e, bl_e, Wg_e, bg_e, Wl_n, bl_n, Wg_n, bg_n):
    src = edge_index[0]
    dst = edge_index[1]

    we = jnp.concatenate([Wl_e, Wg_e], axis=1).astype(jnp.bfloat16)
    wn = jnp.concatenate([Wl_n, Wg_n], axis=1).astype(jnp.bfloat16)
    w1e, w2e, w3e = we[:D], we[D:2 * D], we[2 * D:]
    w1n, w2n, w3n = wn[:D], wn[D:2 * D], wn[2 * D:]
    be = jnp.concatenate([bl_e, bg_e])[None, :]
    bn = jnp.concatenate([bl_n, bg_n])[None, :]

    vi, vj = _sc_gather(node_features, src, dst)
    new_edge, messages = _tc_mlps(vi, vj, edge_features, shared_edge_weights,
                                  w1e, w2e, w3e, be, w1n, w2n, w3n, bn)
    zeros = jnp.zeros((N_NODES, D), jnp.float32)
    acc = _sc_scatter(messages, dst, zeros)
    new_nodes = _tc_axpy(node_features, shared_node_weights, acc[0], acc[1])
    return (new_nodes, new_edge, state_attr)


# speedup vs baseline: 3.7493x; 1.0523x over previous
import functools

import jax
import jax.numpy as jnp
from jax import lax
from jax.experimental import pallas as pl
from jax.experimental.pallas import tpu as pltpu
from jax.experimental.pallas import tpu_sc as plsc

N_NODES = 10000
N_EDGES = 320000
D = 128

NC = 2
NS = 16
NW = NC * NS
EPW = N_EDGES // NW
CHUNK = 400
SCHUNK = 200
SLAB = 624
TAIL = N_NODES - NS * SLAB

_LOG2 = 0.6931471805599453

def _vector_mesh():
    return plsc.VectorSubcoreMesh(core_axis_name="c", subcore_axis_name="s")


def _gather_body(nodes_hbm, src_hbm, dst_hbm, vi_hbm, vj_hbm,
                 idx0, idx1, rows0, rows1, sem0, sem1):
    wid = lax.axis_index("s") * NC + lax.axis_index("c")
    base = wid * EPW
    nchunks = EPW // CHUNK
    bufs = ((src_hbm, vi_hbm, idx0, rows0, sem0),
            (dst_hbm, vj_hbm, idx1, rows1, sem1))

    for tbl, _, idx_v, rows_v, sem in bufs:
        pltpu.sync_copy(tbl.at[pl.ds(base, CHUNK)], idx_v)
        pltpu.make_async_copy(nodes_hbm.at[idx_v], rows_v, sem).start()

    @pl.loop(0, nchunks)
    def _(c):
        off = base + c * CHUNK
        for tbl, out, idx_v, rows_v, sem in bufs:
            pltpu.make_async_copy(nodes_hbm.at[idx_v], rows_v, sem).wait()
            pltpu.sync_copy(rows_v, out.at[pl.ds(off, CHUNK)])

            @pl.when(c + 1 < nchunks)
            def _():
                pltpu.sync_copy(tbl.at[pl.ds(off + CHUNK, CHUNK)], idx_v)
                pltpu.make_async_copy(nodes_hbm.at[idx_v], rows_v, sem).start()


def _sc_gather(node_features, src, dst):
    k = pl.kernel(
        _gather_body,
        out_type=[
            jax.ShapeDtypeStruct((N_EDGES, D), jnp.float32),
            jax.ShapeDtypeStruct((N_EDGES, D), jnp.float32),
        ],
        mesh=_vector_mesh(),
        scratch_types=[
            pltpu.VMEM((CHUNK,), jnp.int32),
            pltpu.VMEM((CHUNK,), jnp.int32),
            pltpu.VMEM((CHUNK, D), jnp.float32),
            pltpu.VMEM((CHUNK, D), jnp.float32),
            pltpu.SemaphoreType.DMA,
            pltpu.SemaphoreType.DMA,
        ],
    )
    return k(node_features, src, dst)


def _scatter_body(msg_hbm, dst_hbm, zeros_hbm, out_hbm, idx_v, rows_v, acc_sh, sem):
    c = lax.axis_index("c")
    s = lax.axis_index("s")
    wid = s * NC + c
    pltpu.sync_copy(zeros_hbm.at[pl.ds(s * SLAB, SLAB)], acc_sh.at[pl.ds(s * SLAB, SLAB)])

    @pl.when(s == NS - 1)
    def _():
        pltpu.sync_copy(zeros_hbm.at[pl.ds(NS * SLAB, TAIL)],
                        acc_sh.at[pl.ds(NS * SLAB, TAIL)])

    plsc.subcore_barrier()
    base = wid * EPW

    @pl.loop(0, EPW, step=SCHUNK)
    def _(off):
        b = base + off
        pltpu.sync_copy(dst_hbm.at[pl.ds(b, SCHUNK)], idx_v)
        pltpu.sync_copy(msg_hbm.at[pl.ds(b, SCHUNK)], rows_v)
        pltpu.sync_copy(rows_v, acc_sh.at[idx_v], add=True)

    plsc.subcore_barrier()
    pltpu.sync_copy(acc_sh.at[pl.ds(s * SLAB, SLAB)],
                    out_hbm.at[c].at[pl.ds(s * SLAB, SLAB)])

    @pl.when(s == NS - 1)
    def _():
        pltpu.sync_copy(acc_sh.at[pl.ds(NS * SLAB, TAIL)],
                        out_hbm.at[c].at[pl.ds(NS * SLAB, TAIL)])


def _sc_scatter(messages, dst, zeros):
    k = pl.kernel(
        _scatter_body,
        out_type=jax.ShapeDtypeStruct((NC, N_NODES, D), jnp.float32),
        mesh=_vector_mesh(),
        scratch_types=[
            pltpu.VMEM((SCHUNK,), jnp.int32),
            pltpu.VMEM((SCHUNK, D), jnp.float32),
            pltpu.VMEM_SHARED((N_NODES, D), jnp.float32),
            pltpu.SemaphoreType.DMA,
        ],
    )
    return k(messages, dst, zeros)


def _mlp_body(vi_ref, vj_ref, e_ref, sew_ref,
              w1e_ref, w2e_ref, w3e_ref, be_ref,
              w1n_ref, w2n_ref, w3n_ref, bn_ref,
              ne_ref, msg_ref):
    vi = vi_ref[...].astype(jnp.bfloat16)
    vj = vj_ref[...].astype(jnp.bfloat16)
    e = e_ref[...]
    e16 = e.astype(jnp.bfloat16)
    pe = (jnp.dot(vi, w1e_ref[...], preferred_element_type=jnp.float32)
          + jnp.dot(e16, w2e_ref[...], preferred_element_type=jnp.float32)
          + jnp.dot(vj, w3e_ref[...], preferred_element_type=jnp.float32)
          + be_ref[...])
    core = jax.nn.softplus(pe[:, :D]) - _LOG2
    gate = jax.nn.sigmoid(pe[:, D:])
    ne = e + core * gate * sew_ref[...]
    ne_ref[...] = ne
    ne16 = ne.astype(jnp.bfloat16)
    pn = (jnp.dot(vi, w1n_ref[...], preferred_element_type=jnp.float32)
          + jnp.dot(vj, w2n_ref[...], preferred_element_type=jnp.float32)
          + jnp.dot(ne16, w3n_ref[...], preferred_element_type=jnp.float32)
          + bn_ref[...])
    msg_ref[...] = (jax.nn.softplus(pn[:, :D]) - _LOG2) * jax.nn.sigmoid(pn[:, D:])


_EB = 1600


def _tc_mlps(vi, vj, e, sew, w1e, w2e, w3e, be, w1n, w2n, w3n, bn):
    grid = (N_EDGES // _EB,)
    row_spec = pl.BlockSpec((_EB, D), lambda i: (i, 0))
    w_spec = pl.BlockSpec((D, 2 * D), lambda i: (0, 0))
    b_spec = pl.BlockSpec((1, 2 * D), lambda i: (0, 0))
    return pl.pallas_call(
        _mlp_body,
        grid=grid,
        in_specs=[row_spec, row_spec, row_spec, row_spec,
                  w_spec, w_spec, w_spec, b_spec,
                  w_spec, w_spec, w_spec, b_spec],
        out_specs=[row_spec, row_spec],
        out_shape=[
            jax.ShapeDtypeStruct((N_EDGES, D), jnp.float32),
            jax.ShapeDtypeStruct((N_EDGES, D), jnp.float32),
        ],
        compiler_params=pltpu.CompilerParams(
            dimension_semantics=("parallel",)),
    )(vi, vj, e, sew, w1e, w2e, w3e, be, w1n, w2n, w3n, bn)


def _axpy_body(n_ref, snw_ref, a0_ref, a1_ref, o_ref):
    o_ref[...] = n_ref[...] + (a0_ref[...] + a1_ref[...]) * snw_ref[...]


def _tc_axpy(nodes, snw, acc0, acc1):
    blk = 1000
    spec = pl.BlockSpec((blk, D), lambda i: (i, 0))
    return pl.pallas_call(
        _axpy_body,
        grid=(N_NODES // blk,),
        in_specs=[spec, spec, spec, spec],
        out_specs=spec,
        out_shape=jax.ShapeDtypeStruct((N_NODES, D), jnp.float32),
    )(nodes, snw, acc0, acc1)


def kernel(node_features, edge_features, edge_index, state_attr,
           shared_node_weights, shared_edge_weights,
           Wl_e, bl_e, Wg_e, bg_e, Wl_n, bl_n, Wg_n, bg_n):
    src = edge_index[0]
    dst = edge_index[1]

    we = jnp.concatenate([Wl_e, Wg_e], axis=1).astype(jnp.bfloat16)
    wn = jnp.concatenate([Wl_n, Wg_n], axis=1).astype(jnp.bfloat16)
    w1e, w2e, w3e = we[:D], we[D:2 * D], we[2 * D:]
    w1n, w2n, w3n = wn[:D], wn[D:2 * D], wn[2 * D:]
    be = jnp.concatenate([bl_e, bg_e])[None, :]
    bn = jnp.concatenate([bl_n, bg_n])[None, :]

    vi, vj = _sc_gather(node_features, src, dst)
    new_edge, messages = _tc_mlps(vi, vj, edge_features, shared_edge_weights,
                                  w1e, w2e, w3e, be, w1n, w2n, w3n, bn)
    zeros = jnp.zeros((N_NODES, D), jnp.float32)
    acc = _sc_scatter(messages, dst, zeros)
    new_nodes = _tc_axpy(node_features, shared_node_weights, acc[0], acc[1])
    return (new_nodes, new_edge, state_attr)


# speedup vs baseline: 4.3178x; 1.1516x over previous
import functools

import jax
import jax.numpy as jnp
from jax import lax
from jax.experimental import pallas as pl
from jax.experimental.pallas import tpu as pltpu
from jax.experimental.pallas import tpu_sc as plsc

N_NODES = 10000
N_EDGES = 320000
D = 128

NC = 2
NS = 16
NW = NC * NS

NCH = 5
CE = N_EDGES // NCH
EPW = CE // NW
CHUNK = 400
SCHUNK = 200
SLAB = 624
TAIL = N_NODES - NS * SLAB

_LOG2 = 0.6931471805599453
_LOG2E = 1.4426950408889634


def _vector_mesh():
    return plsc.VectorSubcoreMesh(core_axis_name="c", subcore_axis_name="s")


def _gather_body(ebase, nodes_hbm, src_hbm, dst_hbm, vi_hbm, vj_hbm,
                 idx0, idx1, rows0, rows1, sem0, sem1):
    wid = lax.axis_index("s") * NC + lax.axis_index("c")
    base = ebase + wid * EPW
    obase = wid * EPW
    nchunks = EPW // CHUNK
    bufs = ((src_hbm, vi_hbm, idx0, rows0, sem0),
            (dst_hbm, vj_hbm, idx1, rows1, sem1))

    for tbl, _, idx_v, rows_v, sem in bufs:
        pltpu.sync_copy(tbl.at[pl.ds(base, CHUNK)], idx_v)
        pltpu.make_async_copy(nodes_hbm.at[idx_v], rows_v, sem).start()

    @pl.loop(0, nchunks)
    def _(c):
        off = c * CHUNK
        for tbl, out, idx_v, rows_v, sem in bufs:
            pltpu.make_async_copy(nodes_hbm.at[idx_v], rows_v, sem).wait()
            pltpu.sync_copy(rows_v, out.at[pl.ds(obase + off, CHUNK)])

            @pl.when(c + 1 < nchunks)
            def _():
                pltpu.sync_copy(tbl.at[pl.ds(base + off + CHUNK, CHUNK)], idx_v)
                pltpu.make_async_copy(nodes_hbm.at[idx_v], rows_v, sem).start()


def _sc_gather(node_features, src, dst, ebase):
    k = pl.kernel(
        functools.partial(_gather_body, ebase),
        out_type=[
            jax.ShapeDtypeStruct((CE, D), jnp.float32),
            jax.ShapeDtypeStruct((CE, D), jnp.float32),
        ],
        mesh=_vector_mesh(),
        scratch_types=[
            pltpu.VMEM((CHUNK,), jnp.int32),
            pltpu.VMEM((CHUNK,), jnp.int32),
            pltpu.VMEM((CHUNK, D), jnp.float32),
            pltpu.VMEM((CHUNK, D), jnp.float32),
            pltpu.SemaphoreType.DMA,
            pltpu.SemaphoreType.DMA,
        ],
    )
    return k(node_features, src, dst)


def _scatter_body(ebase, msg_hbm, dst_hbm, zeros_hbm, out_hbm,
                  idx_v, rows_v, acc_sh, sem):
    c = lax.axis_index("c")
    s = lax.axis_index("s")
    wid = s * NC + c
    pltpu.sync_copy(zeros_hbm.at[pl.ds(s * SLAB, SLAB)],
                    acc_sh.at[pl.ds(s * SLAB, SLAB)])

    @pl.when(s == NS - 1)
    def _():
        pltpu.sync_copy(zeros_hbm.at[pl.ds(NS * SLAB, TAIL)],
                        acc_sh.at[pl.ds(NS * SLAB, TAIL)])

    plsc.subcore_barrier()

    @pl.loop(0, EPW, step=SCHUNK)
    def _(off):
        pltpu.sync_copy(dst_hbm.at[pl.ds(ebase + wid * EPW + off, SCHUNK)], idx_v)
        pltpu.sync_copy(msg_hbm.at[pl.ds(wid * EPW + off, SCHUNK)], rows_v)
        pltpu.sync_copy(rows_v, acc_sh.at[idx_v], add=True)

    plsc.subcore_barrier()
    pltpu.sync_copy(acc_sh.at[pl.ds(s * SLAB, SLAB)],
                    out_hbm.at[c].at[pl.ds(s * SLAB, SLAB)])

    @pl.when(s == NS - 1)
    def _():
        pltpu.sync_copy(acc_sh.at[pl.ds(NS * SLAB, TAIL)],
                        out_hbm.at[c].at[pl.ds(NS * SLAB, TAIL)])


def _sc_scatter(messages, dst, zeros, ebase):
    k = pl.kernel(
        functools.partial(_scatter_body, ebase),
        out_type=jax.ShapeDtypeStruct((NC, N_NODES, D), jnp.float32),
        mesh=_vector_mesh(),
        scratch_types=[
            pltpu.VMEM((SCHUNK,), jnp.int32),
            pltpu.VMEM((SCHUNK, D), jnp.float32),
            pltpu.VMEM_SHARED((N_NODES, D), jnp.float32),
            pltpu.SemaphoreType.DMA,
        ],
    )
    return k(messages, dst, zeros)


def _sigmoid(x):
    return 0.5 * jnp.tanh(0.5 * x) + 0.5


def _softplus2(x):
    t = jnp.exp2(-jnp.abs(x) * _LOG2E)
    return jnp.maximum(x, 0.0) + _LOG2 * (jnp.log2(1.0 + t) - 1.0)


def _mlp_body(vi_ref, vj_ref, e_ref, sew_ref,
              we_ref, be_ref, wn_ref, bn_ref,
              ne_ref, msg_ref, x_ref):
    e = e_ref[...]
    x_ref[:, :D] = vi_ref[...].astype(jnp.bfloat16)
    x_ref[:, D:2 * D] = vj_ref[...].astype(jnp.bfloat16)
    x_ref[:, 2 * D:] = e.astype(jnp.bfloat16)
    pe = jnp.dot(x_ref[...], we_ref[...],
                 preferred_element_type=jnp.float32) + be_ref[...]
    core = _softplus2(pe[:, :D])
    gate = _sigmoid(pe[:, D:])
    ne = e + core * gate * sew_ref[...]
    ne_ref[...] = ne
    x_ref[:, 2 * D:] = ne.astype(jnp.bfloat16)
    pn = jnp.dot(x_ref[...], wn_ref[...],
                 preferred_element_type=jnp.float32) + bn_ref[...]
    msg_ref[...] = _softplus2(pn[:, :D]) * _sigmoid(pn[:, D:])


_EB = 3200


def _tc_mlps(vi, vj, e, sew, we, be, wn, bn, cidx):
    grid = (CE // _EB,)
    blk0 = cidx * (CE // _EB)
    chunk_spec = pl.BlockSpec((_EB, D), lambda i: (i, 0))
    full_spec = pl.BlockSpec((_EB, D), lambda i: (blk0 + i, 0))
    w_spec = pl.BlockSpec((3 * D, 2 * D), lambda i: (0, 0))
    b_spec = pl.BlockSpec((1, 2 * D), lambda i: (0, 0))
    return pl.pallas_call(
        _mlp_body,
        grid=grid,
        in_specs=[chunk_spec, chunk_spec, full_spec, full_spec,
                  w_spec, b_spec, w_spec, b_spec],
        out_specs=[chunk_spec, chunk_spec],
        out_shape=[
            jax.ShapeDtypeStruct((CE, D), jnp.float32),
            jax.ShapeDtypeStruct((CE, D), jnp.float32),
        ],
        scratch_shapes=[pltpu.VMEM((_EB, 3 * D), jnp.bfloat16)],
    )(vi, vj, e, sew, we, be, wn, bn)


def _axpy_body(*refs):
    n_ref, snw_ref = refs[0], refs[1]
    accs = refs[2:-1]
    o_ref = refs[-1]
    tot = accs[0][...]
    for a in accs[1:]:
        tot = tot + a[...]
    o_ref[...] = n_ref[...] + tot * snw_ref[...]


def _tc_axpy(nodes, snw, accs):
    blk = 1000
    spec = pl.BlockSpec((blk, D), lambda i: (i, 0))
    return pl.pallas_call(
        _axpy_body,
        grid=(N_NODES // blk,),
        in_specs=[spec, spec] + [spec] * len(accs),
        out_specs=spec,
        out_shape=jax.ShapeDtypeStruct((N_NODES, D), jnp.float32),
    )(nodes, snw, *accs)


def kernel(node_features, edge_features, edge_index, state_attr,
           shared_node_weights, shared_edge_weights,
           Wl_e, bl_e, Wg_e, bg_e, Wl_n, bl_n, Wg_n, bg_n):
    src = edge_index[0]
    dst = edge_index[1]

    we = jnp.concatenate([Wl_e, Wg_e], axis=1).astype(jnp.bfloat16)
    we = jnp.concatenate([we[:D], we[2 * D:], we[D:2 * D]], axis=0)
    wn = jnp.concatenate([Wl_n, Wg_n], axis=1).astype(jnp.bfloat16)
    be = jnp.concatenate([bl_e, bg_e])[None, :]
    bn = jnp.concatenate([bl_n, bg_n])[None, :]
    zeros = jnp.zeros((N_NODES, D), jnp.float32)

    new_edges = []
    accs = []
    for c in range(NCH):
        vi, vj = _sc_gather(node_features, src, dst, c * CE)
        ne_c, msg_c = _tc_mlps(vi, vj, edge_features, shared_edge_weights,
                               we, be, wn, bn, c)
        acc_c = _sc_scatter(msg_c, dst, zeros, c * CE)
        new_edges.append(ne_c)
        accs.append(acc_c[0])
        accs.append(acc_c[1])

    new_edge = jnp.concatenate(new_edges, axis=0)
    new_nodes = _tc_axpy(node_features, shared_node_weights, accs)
    return (new_nodes, new_edge, state_attr)


# speedup vs baseline: 4.4143x; 1.0224x over previous
import functools

import jax
import jax.numpy as jnp
from jax import lax
from jax.experimental import pallas as pl
from jax.experimental.pallas import tpu as pltpu
from jax.experimental.pallas import tpu_sc as plsc

N_NODES = 10000
N_EDGES = 320000
D = 128

NC = 2
NS = 16
NW = NC * NS

NCH = 5
CE = N_EDGES // NCH
EPW = CE // NW
CHUNK = 400
SCHUNK = 200
SLAB = 624
TAIL = N_NODES - NS * SLAB

_LOG2 = 0.6931471805599453
_LOG2E = 1.4426950408889634


def _vector_mesh():
    return plsc.VectorSubcoreMesh(core_axis_name="c", subcore_axis_name="s")


def _gather_body(ebase, nodes_hbm, src_hbm, dst_hbm, vi_hbm, vj_hbm,
                 idx0, idx1, rows0, rows1, sem0, sem1):
    wid = lax.axis_index("s") * NC + lax.axis_index("c")
    base = ebase + wid * EPW
    obase = wid * EPW
    nchunks = EPW // CHUNK
    bufs = ((src_hbm, vi_hbm, idx0, rows0, sem0),
            (dst_hbm, vj_hbm, idx1, rows1, sem1))

    for tbl, _, idx_v, rows_v, sem in bufs:
        pltpu.sync_copy(tbl.at[pl.ds(base, CHUNK)], idx_v)
        pltpu.make_async_copy(nodes_hbm.at[idx_v], rows_v, sem).start()

    @pl.loop(0, nchunks)
    def _(c):
        off = c * CHUNK
        for tbl, out, idx_v, rows_v, sem in bufs:
            pltpu.make_async_copy(nodes_hbm.at[idx_v], rows_v, sem).wait()
            pltpu.sync_copy(rows_v, out.at[pl.ds(obase + off, CHUNK)])

            @pl.when(c + 1 < nchunks)
            def _():
                pltpu.sync_copy(tbl.at[pl.ds(base + off + CHUNK, CHUNK)], idx_v)
                pltpu.make_async_copy(nodes_hbm.at[idx_v], rows_v, sem).start()


def _sc_gather(node_features, src, dst, ebase):
    k = pl.kernel(
        functools.partial(_gather_body, ebase),
        out_type=[
            jax.ShapeDtypeStruct((CE, D), jnp.float32),
            jax.ShapeDtypeStruct((CE, D), jnp.float32),
        ],
        mesh=_vector_mesh(),
        scratch_types=[
            pltpu.VMEM((CHUNK,), jnp.int32),
            pltpu.VMEM((CHUNK,), jnp.int32),
            pltpu.VMEM((CHUNK, D), jnp.float32),
            pltpu.VMEM((CHUNK, D), jnp.float32),
            pltpu.SemaphoreType.DMA,
            pltpu.SemaphoreType.DMA,
        ],
    )
    return k(node_features, src, dst)


def _scatter_body(ebase, msg_hbm, dst_hbm, zeros_hbm, out_hbm,
                  idx_v, rows_v, acc_sh, sem):
    c = lax.axis_index("c")
    s = lax.axis_index("s")
    wid = s * NC + c
    pltpu.sync_copy(zeros_hbm.at[pl.ds(s * SLAB, SLAB)],
                    acc_sh.at[pl.ds(s * SLAB, SLAB)])

    @pl.when(s == NS - 1)
    def _():
        pltpu.sync_copy(zeros_hbm.at[pl.ds(NS * SLAB, TAIL)],
                        acc_sh.at[pl.ds(NS * SLAB, TAIL)])

    plsc.subcore_barrier()

    @pl.loop(0, EPW, step=SCHUNK)
    def _(off):
        pltpu.sync_copy(dst_hbm.at[pl.ds(ebase + wid * EPW + off, SCHUNK)], idx_v)
        pltpu.sync_copy(msg_hbm.at[pl.ds(wid * EPW + off, SCHUNK)], rows_v)
        pltpu.sync_copy(rows_v, acc_sh.at[idx_v], add=True)

    plsc.subcore_barrier()
    pltpu.sync_copy(acc_sh.at[pl.ds(s * SLAB, SLAB)],
                    out_hbm.at[c].at[pl.ds(s * SLAB, SLAB)])

    @pl.when(s == NS - 1)
    def _():
        pltpu.sync_copy(acc_sh.at[pl.ds(NS * SLAB, TAIL)],
                        out_hbm.at[c].at[pl.ds(NS * SLAB, TAIL)])


def _sc_scatter(messages, dst, zeros, ebase):
    k = pl.kernel(
        functools.partial(_scatter_body, ebase),
        out_type=jax.ShapeDtypeStruct((NC, N_NODES, D), jnp.float32),
        mesh=_vector_mesh(),
        scratch_types=[
            pltpu.VMEM((SCHUNK,), jnp.int32),
            pltpu.VMEM((SCHUNK, D), jnp.float32),
            pltpu.VMEM_SHARED((N_NODES, D), jnp.float32),
            pltpu.SemaphoreType.DMA,
        ],
    )
    return k(messages, dst, zeros)


def _sigmoid(x):
    return 0.5 * jnp.tanh(0.5 * x) + 0.5


def _softplus2(x):
    t = jnp.exp2(-jnp.abs(x) * _LOG2E)
    return jnp.maximum(x, 0.0) + _LOG2 * (jnp.log2(1.0 + t) - 1.0)


def _mlp_body(vi_ref, vj_ref, e_ref, sew_ref,
              we_ref, be_ref, wn_ref, bn_ref,
              ne_in_ref, ne_ref, msg_ref, x_ref):
    del ne_in_ref
    e = e_ref[...]
    x_ref[:, :D] = vi_ref[...].astype(jnp.bfloat16)
    x_ref[:, D:2 * D] = vj_ref[...].astype(jnp.bfloat16)
    x_ref[:, 2 * D:] = e.astype(jnp.bfloat16)
    pe = jnp.dot(x_ref[...], we_ref[...],
                 preferred_element_type=jnp.float32) + be_ref[...]
    core = _softplus2(pe[:, :D])
    gate = _sigmoid(pe[:, D:])
    ne = e + core * gate * sew_ref[...]
    ne_ref[...] = ne
    x_ref[:, 2 * D:] = ne.astype(jnp.bfloat16)
    pn = jnp.dot(x_ref[...], wn_ref[...],
                 preferred_element_type=jnp.float32) + bn_ref[...]
    msg_ref[...] = _softplus2(pn[:, :D]) * _sigmoid(pn[:, D:])


_EB = 3200


def _tc_mlps(vi, vj, e, sew, we, be, wn, bn, cidx, ne_buf):
    grid = (CE // _EB,)
    blk0 = cidx * (CE // _EB)
    packed_spec = pl.BlockSpec((_EB, D), lambda i: (i, 0))
    chunk_spec = pl.BlockSpec((_EB, D), lambda i: (i, 0))
    full_spec = pl.BlockSpec((_EB, D), lambda i: (blk0 + i, 0))
    dummy_spec = pl.BlockSpec((8, D), lambda i: (0, 0))
    w_spec = pl.BlockSpec((3 * D, 2 * D), lambda i: (0, 0))
    b_spec = pl.BlockSpec((1, 2 * D), lambda i: (0, 0))
    return pl.pallas_call(
        _mlp_body,
        grid=grid,
        in_specs=[packed_spec, packed_spec, full_spec, full_spec,
                  w_spec, b_spec, w_spec, b_spec, dummy_spec],
        out_specs=[full_spec, chunk_spec],
        out_shape=[
            jax.ShapeDtypeStruct((N_EDGES, D), jnp.float32),
            jax.ShapeDtypeStruct((CE, D), jnp.float32),
        ],
        scratch_shapes=[pltpu.VMEM((_EB, 3 * D), jnp.bfloat16)],
        input_output_aliases={8: 0},
    )(vi, vj, e, sew, we, be, wn, bn, ne_buf)


def _axpy_body(*refs):
    n_ref, snw_ref = refs[0], refs[1]
    accs = refs[2:-1]
    o_ref = refs[-1]
    tot = accs[0][...]
    for a in accs[1:]:
        tot = tot + a[...]
    o_ref[...] = n_ref[...] + tot * snw_ref[...]


def _tc_axpy(nodes, snw, accs):
    blk = 1000
    spec = pl.BlockSpec((blk, D), lambda i: (i, 0))
    return pl.pallas_call(
        _axpy_body,
        grid=(N_NODES // blk,),
        in_specs=[spec, spec] + [spec] * len(accs),
        out_specs=spec,
        out_shape=jax.ShapeDtypeStruct((N_NODES, D), jnp.float32),
    )(nodes, snw, *accs)


def kernel(node_features, edge_features, edge_index, state_attr,
           shared_node_weights, shared_edge_weights,
           Wl_e, bl_e, Wg_e, bg_e, Wl_n, bl_n, Wg_n, bg_n):
    src = edge_index[0]
    dst = edge_index[1]

    we = jnp.concatenate([Wl_e, Wg_e], axis=1).astype(jnp.bfloat16)
    we = jnp.concatenate([we[:D], we[2 * D:], we[D:2 * D]], axis=0)
    wn = jnp.concatenate([Wl_n, Wg_n], axis=1).astype(jnp.bfloat16)
    be = jnp.concatenate([bl_e, bg_e])[None, :]
    bn = jnp.concatenate([bl_n, bg_n])[None, :]
    zeros = jnp.zeros((N_NODES, D), jnp.float32)

    nf_packed = node_features

    ne_buf = jnp.zeros((N_EDGES, D), jnp.float32)
    accs = []
    for c in range(NCH):
        vi, vj = _sc_gather(nf_packed, src, dst, c * CE)
        ne_buf, msg_c = _tc_mlps(vi, vj, edge_features, shared_edge_weights,
                                 we, be, wn, bn, c, ne_buf)
        acc_c = _sc_scatter(msg_c, dst, zeros, c * CE)
        accs.append(acc_c[0])
        accs.append(acc_c[1])

    new_nodes = _tc_axpy(node_features, shared_node_weights, accs)
    return (new_nodes, ne_buf, state_attr)


# speedup vs baseline: 4.8269x; 1.0935x over previous
import functools

import jax
import jax.numpy as jnp
from jax import lax
from jax.experimental import pallas as pl
from jax.experimental.pallas import tpu as pltpu
from jax.experimental.pallas import tpu_sc as plsc

N_NODES = 10000
N_EDGES = 320000
D = 128

NC = 2
NS = 16
NW = NC * NS

NCH = 5
CE = N_EDGES // NCH
EPW = CE // NW
CHUNK = 400
SCHUNK = 200
SLAB = 624
TAIL = N_NODES - NS * SLAB

_LOG2 = 0.6931471805599453
_LOG2E = 1.4426950408889634


def _vector_mesh():
    return plsc.VectorSubcoreMesh(core_axis_name="c", subcore_axis_name="s")


def _gather_body(ebase, nodes_hbm, src_hbm, dst_hbm, vi_hbm, vj_hbm,
                 idx0, idx1, rows0, rows1, sem0, sem1):
    wid = lax.axis_index("s") * NC + lax.axis_index("c")
    base = ebase + wid * EPW
    obase = wid * EPW
    nchunks = EPW // CHUNK
    bufs = ((src_hbm, vi_hbm, idx0, rows0, sem0),
            (dst_hbm, vj_hbm, idx1, rows1, sem1))

    for tbl, _, idx_v, rows_v, sem in bufs:
        pltpu.sync_copy(tbl.at[pl.ds(base, CHUNK)], idx_v)
        pltpu.make_async_copy(nodes_hbm.at[idx_v], rows_v, sem).start()

    @pl.loop(0, nchunks)
    def _(c):
        off = c * CHUNK
        for tbl, out, idx_v, rows_v, sem in bufs:
            pltpu.make_async_copy(nodes_hbm.at[idx_v], rows_v, sem).wait()
            pltpu.sync_copy(rows_v, out.at[pl.ds(obase + off, CHUNK)])

            @pl.when(c + 1 < nchunks)
            def _():
                pltpu.sync_copy(tbl.at[pl.ds(base + off + CHUNK, CHUNK)], idx_v)
                pltpu.make_async_copy(nodes_hbm.at[idx_v], rows_v, sem).start()


def _sc_gather(node_features, src, dst, ebase):
    k = pl.kernel(
        functools.partial(_gather_body, ebase),
        out_type=[
            jax.ShapeDtypeStruct((CE, D), jnp.float32),
            jax.ShapeDtypeStruct((CE, D), jnp.float32),
        ],
        mesh=_vector_mesh(),
        scratch_types=[
            pltpu.VMEM((CHUNK,), jnp.int32),
            pltpu.VMEM((CHUNK,), jnp.int32),
            pltpu.VMEM((CHUNK, D), jnp.float32),
            pltpu.VMEM((CHUNK, D), jnp.float32),
            pltpu.SemaphoreType.DMA,
            pltpu.SemaphoreType.DMA,
        ],
    )
    return k(node_features, src, dst)


def _scatter_body(ebase, msg_hbm, dst_hbm, zeros_hbm, out_hbm,
                  idx_v, rows_v, acc_sh, sem):
    c = lax.axis_index("c")
    s = lax.axis_index("s")
    wid = s * NC + c
    pltpu.sync_copy(zeros_hbm.at[pl.ds(s * SLAB, SLAB)],
                    acc_sh.at[pl.ds(s * SLAB, SLAB)])

    @pl.when(s == NS - 1)
    def _():
        pltpu.sync_copy(zeros_hbm.at[pl.ds(NS * SLAB, TAIL)],
                        acc_sh.at[pl.ds(NS * SLAB, TAIL)])

    plsc.subcore_barrier()

    @pl.loop(0, EPW, step=SCHUNK)
    def _(off):
        pltpu.sync_copy(dst_hbm.at[pl.ds(ebase + wid * EPW + off, SCHUNK)], idx_v)
        pltpu.sync_copy(msg_hbm.at[pl.ds(wid * EPW + off, SCHUNK)], rows_v)
        pltpu.sync_copy(rows_v, acc_sh.at[idx_v], add=True)

    plsc.subcore_barrier()
    pltpu.sync_copy(acc_sh.at[pl.ds(s * SLAB, SLAB)],
                    out_hbm.at[c].at[pl.ds(s * SLAB, SLAB)])

    @pl.when(s == NS - 1)
    def _():
        pltpu.sync_copy(acc_sh.at[pl.ds(NS * SLAB, TAIL)],
                        out_hbm.at[c].at[pl.ds(NS * SLAB, TAIL)])


def _sc_scatter(messages, dst, zeros, ebase):
    k = pl.kernel(
        functools.partial(_scatter_body, ebase),
        out_type=jax.ShapeDtypeStruct((NC, N_NODES, D), jnp.float32),
        mesh=_vector_mesh(),
        scratch_types=[
            pltpu.VMEM((SCHUNK,), jnp.int32),
            pltpu.VMEM((SCHUNK, D), jnp.float32),
            pltpu.VMEM_SHARED((N_NODES, D), jnp.float32),
            pltpu.SemaphoreType.DMA,
        ],
    )
    return k(messages, dst, zeros)


def _sigmoid(x):
    return 0.5 * jnp.tanh(0.5 * x) + 0.5


def _softplus2(x):
    t = jnp.exp2(-jnp.abs(x) * _LOG2E)
    return jnp.maximum(x, 0.0) + _LOG2 * (jnp.log2(1.0 + t) - 1.0)


def _mlp_body(vi_ref, vj_ref, e_ref, sew_ref,
              we_ref, be_ref, wn_ref, bn_ref,
              ne_in_ref, ne_ref, msg_ref, x_ref):
    del ne_in_ref
    e = e_ref[...]
    x_ref[:, :D] = vi_ref[...].astype(jnp.bfloat16)
    x_ref[:, D:2 * D] = vj_ref[...].astype(jnp.bfloat16)
    x_ref[:, 2 * D:] = e.astype(jnp.bfloat16)
    pe = jnp.dot(x_ref[...], we_ref[...],
                 preferred_element_type=jnp.float32) + be_ref[...]
    core = _softplus2(pe[:, :D])
    gate = _sigmoid(pe[:, D:])
    ne = e + core * gate * sew_ref[...]
    ne_ref[...] = ne
    x_ref[:, 2 * D:] = ne.astype(jnp.bfloat16)
    pn = jnp.dot(x_ref[...], wn_ref[...],
                 preferred_element_type=jnp.float32) + bn_ref[...]
    msg_ref[...] = _softplus2(pn[:, :D]) * _sigmoid(pn[:, D:])


def _mlp_body_first(vi_ref, vj_ref, e_ref, sew_ref,
                    we_ref, be_ref, wn_ref, bn_ref,
                    ne_ref, msg_ref, x_ref):
    _mlp_body(vi_ref, vj_ref, e_ref, sew_ref,
              we_ref, be_ref, wn_ref, bn_ref,
              None, ne_ref, msg_ref, x_ref)


_EB = 3200


def _tc_mlps(vi, vj, e, sew, we, be, wn, bn, cidx, ne_buf):
    grid = (CE // _EB,)
    blk0 = cidx * (CE // _EB)
    packed_spec = pl.BlockSpec((_EB, D), lambda i: (i, 0))
    chunk_spec = pl.BlockSpec((_EB, D), lambda i: (i, 0))
    full_spec = pl.BlockSpec((_EB, D), lambda i: (blk0 + i, 0))
    dummy_spec = pl.BlockSpec((8, D), lambda i: (0, 0))
    w_spec = pl.BlockSpec((3 * D, 2 * D), lambda i: (0, 0))
    b_spec = pl.BlockSpec((1, 2 * D), lambda i: (0, 0))
    in_specs = [packed_spec, packed_spec, full_spec, full_spec,
                w_spec, b_spec, w_spec, b_spec]
    args = (vi, vj, e, sew, we, be, wn, bn)
    aliases = {}
    if ne_buf is not None:
        in_specs = in_specs + [dummy_spec]
        args = args + (ne_buf,)
        aliases = {8: 0}
    body = _mlp_body if ne_buf is not None else _mlp_body_first
    return pl.pallas_call(
        body,
        grid=grid,
        in_specs=in_specs,
        out_specs=[full_spec, chunk_spec],
        out_shape=[
            jax.ShapeDtypeStruct((N_EDGES, D), jnp.float32),
            jax.ShapeDtypeStruct((CE, D), jnp.float32),
        ],
        scratch_shapes=[pltpu.VMEM((_EB, 3 * D), jnp.bfloat16)],
        input_output_aliases=aliases,
    )(*args)


def _axpy_body(*refs):
    n_ref, snw_ref = refs[0], refs[1]
    accs = refs[2:-1]
    o_ref = refs[-1]
    tot = accs[0][...]
    for a in accs[1:]:
        tot = tot + a[...]
    o_ref[...] = n_ref[...] + tot * snw_ref[...]


def _tc_axpy(nodes, snw, accs):
    blk = 1000
    spec = pl.BlockSpec((blk, D), lambda i: (i, 0))
    return pl.pallas_call(
        _axpy_body,
        grid=(N_NODES // blk,),
        in_specs=[spec, spec] + [spec] * len(accs),
        out_specs=spec,
        out_shape=jax.ShapeDtypeStruct((N_NODES, D), jnp.float32),
    )(nodes, snw, *accs)


def kernel(node_features, edge_features, edge_index, state_attr,
           shared_node_weights, shared_edge_weights,
           Wl_e, bl_e, Wg_e, bg_e, Wl_n, bl_n, Wg_n, bg_n):
    src = edge_index[0]
    dst = edge_index[1]

    we = jnp.concatenate([Wl_e, Wg_e], axis=1).astype(jnp.bfloat16)
    we = jnp.concatenate([we[:D], we[2 * D:], we[D:2 * D]], axis=0)
    wn = jnp.concatenate([Wl_n, Wg_n], axis=1).astype(jnp.bfloat16)
    be = jnp.concatenate([bl_e, bg_e])[None, :]
    bn = jnp.concatenate([bl_n, bg_n])[None, :]
    zeros = jnp.zeros((N_NODES, D), jnp.float32)

    nf_packed = node_features

    ne_buf = None
    accs = []
    for c in range(NCH):
        vi, vj = _sc_gather(nf_packed, src, dst, c * CE)
        ne_buf, msg_c = _tc_mlps(vi, vj, edge_features, shared_edge_weights,
                                 we, be, wn, bn, c, ne_buf)
        acc_c = _sc_scatter(msg_c, dst, zeros, c * CE)
        accs.append(acc_c[0])
        accs.append(acc_c[1])

    new_nodes = _tc_axpy(node_features, shared_node_weights, accs)
    return (new_nodes, ne_buf, state_attr)
